# baseline (speedup 1.0000x reference)
import functools

import jax
import jax.numpy as jnp
from jax import lax
from jax.experimental import pallas as pl
from jax.experimental.pallas import tpu as pltpu
from jax.experimental.pallas import tpu_sc as plsc

VOCAB = 5
EMB = 20
ENT = 3
BATCH = 16384
SEQ = 200

NC = 2
NS = 16
LANES = 16
NW = NC * NS

LANE_W = BATCH // NW
KGROUPS = LANE_W // LANES
NCH = (SEQ + 15) // 16


def _sc_body(idx_hbm, emb_hbm, wt_hbm, b_hbm, out_hbm,
             emb_v, wt_v, b_v, ib0, ib1, ob0, ob1,
             sin0, sin1, sout0, sout1):
    wid = lax.axis_index("s") * NC + lax.axis_index("c")
    lane0 = pl.multiple_of(wid * LANE_W, 128)

    ibufs = (ib0, ib1)
    obufs = (ob0, ob1)
    sins = (sin0, sin1)
    souts = (sout0, sout1)

    def rows_of(g):
        return min(16, SEQ - g * 16)

    def in_copy(g):
        r = rows_of(g)
        return pltpu.make_async_copy(
            idx_hbm.at[pl.ds(g * 16, r), pl.ds(lane0, LANE_W)],
            ibufs[g % 2].at[pl.ds(0, r)], sins[g % 2])

    def out_copy(g):
        r = rows_of(g)
        return pltpu.make_async_copy(
            obufs[g % 2].at[:, pl.ds(0, r)],
            out_hbm.at[:, pl.ds(g * 16, r), pl.ds(lane0, LANE_W)],
            souts[g % 2])

    in_copy(0).start()
    in_copy(1).start()

    pltpu.sync_copy(emb_hbm, emb_v)
    pltpu.sync_copy(wt_hbm, wt_v)
    pltpu.sync_copy(b_hbm, b_v)
    lanes = lax.iota(jnp.int32, 16)
    cols = [plsc.load_gather(emb_v, [lanes, jnp.full((16,), d, jnp.int32)])
            for d in range(EMB)]
    logits = []
    for c in range(ENT):
        acc = plsc.load_gather(b_v, [jnp.full((16,), c, jnp.int32)])
        for d in range(EMB):
            wcd = plsc.load_gather(
                wt_v, [jnp.full((16,), c, jnp.int32),
                       jnp.full((16,), d, jnp.int32)])
            acc = acc + cols[d] * wcd
        logits.append(acc)
    m = jnp.maximum(logits[0], jnp.maximum(logits[1], logits[2]))
    exps = [jnp.exp(l - m) for l in logits]
    denom = exps[0] + exps[1] + exps[2]
    tbl = [e / denom for e in exps]

    for g in range(NCH):
        in_copy(g).wait()
        if g >= 2:
            out_copy(g - 2).wait()

        ibuf = ibufs[g % 2]
        obuf = obufs[g % 2]
        nrows = rows_of(g)

        def step(j, carry):
            col = j * LANES
            for r in range(nrows):
                idx = ibuf[r, pl.ds(col, LANES)]
                for c in range(ENT):
                    obuf[c, r, pl.ds(col, LANES)] = tbl[c].at[idx].get(
                        mode="promise_in_bounds")
            return carry

        lax.fori_loop(0, KGROUPS, step, 0, unroll=False)
        out_copy(g).start()
        if g + 2 < NCH:
            in_copy(g + 2).start()

    out_copy(NCH - 2).wait()
    out_copy(NCH - 1).wait()


@jax.jit
def _run(idx_t, emb_table, wt, b):
    mesh = plsc.VectorSubcoreMesh(core_axis_name="c", subcore_axis_name="s")
    fn = functools.partial(
        pl.kernel,
        out_type=jax.ShapeDtypeStruct((ENT, SEQ, BATCH), jnp.float32),
        mesh=mesh,
        compiler_params=pltpu.CompilerParams(needs_layout_passes=False),
        scratch_types=[
            pltpu.VMEM((VOCAB, EMB), jnp.float32),
            pltpu.VMEM((ENT, EMB), jnp.float32),
            pltpu.VMEM((ENT,), jnp.float32),
            pltpu.VMEM((16, LANE_W), jnp.int32),
            pltpu.VMEM((16, LANE_W), jnp.int32),
            pltpu.VMEM((ENT, 16, LANE_W), jnp.float32),
            pltpu.VMEM((ENT, 16, LANE_W), jnp.float32),
            pltpu.SemaphoreType.DMA,
            pltpu.SemaphoreType.DMA,
            pltpu.SemaphoreType.DMA,
            pltpu.SemaphoreType.DMA,
        ],
    )(_sc_body)
    return fn(idx_t, emb_table, wt, b)


def kernel(inputs, emb_table, W, b):
    idx_t = inputs.astype(jnp.int32).T
    out3 = _run(idx_t, emb_table.astype(jnp.float32),
                W.astype(jnp.float32).T, b.astype(jnp.float32))
    return out3.transpose(2, 1, 0)


# speedup vs baseline: 1.0572x; 1.0572x over previous
import functools

import jax
import jax.numpy as jnp
from jax import lax
from jax.experimental import pallas as pl
from jax.experimental.pallas import tpu as pltpu
from jax.experimental.pallas import tpu_sc as plsc

VOCAB = 5
EMB = 20
ENT = 3
BATCH = 16384
SEQ = 200

NC = 2
NS = 16
LANES = 16
NW = NC * NS

LANE_W = BATCH // NW
KGROUPS = LANE_W // LANES
NCH = (SEQ + 15) // 16

P_EMBT = 0
P_W = EMB
P_B = EMB + ENT * EMB
P_ROWS = P_B + ENT


def _sc_body(idx_hbm, par_hbm, out_hbm,
             par_v, ib0, ib1, ob0, ob1,
             sin0, sin1, sout0, sout1):
    wid = lax.axis_index("s") * NC + lax.axis_index("c")
    lane0 = pl.multiple_of(wid * LANE_W, 128)

    ibufs = (ib0, ib1)
    obufs = (ob0, ob1)
    sins = (sin0, sin1)
    souts = (sout0, sout1)

    def rows_of(g):
        return min(16, SEQ - g * 16)

    def in_copy(g):
        r = rows_of(g)
        return pltpu.make_async_copy(
            idx_hbm.at[pl.ds(g * 16, r), pl.ds(lane0, LANE_W)],
            ibufs[g % 2].at[pl.ds(0, r)], sins[g % 2])

    def out_copy(g):
        r = rows_of(g)
        return pltpu.make_async_copy(
            obufs[g % 2].at[:, pl.ds(0, r)],
            out_hbm.at[:, pl.ds(g * 16, r), pl.ds(lane0, LANE_W)],
            souts[g % 2])

    in_copy(0).start()
    in_copy(1).start()

    pltpu.sync_copy(par_hbm, par_v)
    logits = []
    for c in range(ENT):
        acc = par_v[P_B + c]
        for d in range(EMB):
            acc = acc + par_v[P_EMBT + d] * par_v[P_W + c * EMB + d]
        logits.append(acc)
    m = jnp.maximum(logits[0], jnp.maximum(logits[1], logits[2]))
    exps = [jnp.exp(l - m) for l in logits]
    denom = exps[0] + exps[1] + exps[2]
    tbl = [e / denom for e in exps]

    def dyn_in_copy(g, parity, nrows=16):
        return pltpu.make_async_copy(
            idx_hbm.at[pl.ds(g * 16, nrows), pl.ds(lane0, LANE_W)],
            ibufs[parity].at[pl.ds(0, nrows)], sins[parity])

    def dyn_out_copy(g, parity, nrows=16):
        return pltpu.make_async_copy(
            obufs[parity].at[:, pl.ds(0, nrows)],
            out_hbm.at[:, pl.ds(g * 16, nrows), pl.ds(lane0, LANE_W)],
            souts[parity])

    def compute(ibuf, obuf, nrows):
        def step(j, carry):
            col = j * LANES
            for r in range(nrows):
                idx = ibuf[r, pl.ds(col, LANES)]
                for c in range(ENT):
                    obuf[c, r, pl.ds(col, LANES)] = tbl[c].at[idx].get(
                        mode="promise_in_bounds")
            return carry

        lax.fori_loop(0, KGROUPS, step, 0, unroll=False)

    dyn_in_copy(0, 0).wait()
    compute(ibufs[0], obufs[0], 16)
    dyn_out_copy(0, 0).start()
    dyn_in_copy(2, 0).start()

    dyn_in_copy(1, 1).wait()
    compute(ibufs[1], obufs[1], 16)
    dyn_out_copy(1, 1).start()
    dyn_in_copy(3, 1).start()

    def pair(p, carry):
        g0 = p * 2
        for parity, g in ((0, g0), (1, g0 + 1)):
            dyn_in_copy(g, parity).wait()
            dyn_out_copy(g - 2, parity).wait()
            compute(ibufs[parity], obufs[parity], 16)
            dyn_out_copy(g, parity).start()
            dyn_in_copy(g + 2, parity).start()
        return carry

    lax.fori_loop(1, 5, pair, 0, unroll=False)

    dyn_in_copy(10, 0).wait()
    dyn_out_copy(8, 0).wait()
    compute(ibufs[0], obufs[0], 16)
    dyn_out_copy(10, 0).start()
    dyn_in_copy(12, 0, 8).start()
    dyn_in_copy(11, 1).wait()
    dyn_out_copy(9, 1).wait()
    compute(ibufs[1], obufs[1], 16)
    dyn_out_copy(11, 1).start()
    dyn_in_copy(12, 0, 8).wait()
    dyn_out_copy(10, 0).wait()
    compute(ibufs[0], obufs[0], 8)
    dyn_out_copy(12, 0, 8).start()

    dyn_out_copy(11, 1).wait()
    dyn_out_copy(12, 0, 8).wait()


@jax.jit
def _run(idx_t, params):
    mesh = plsc.VectorSubcoreMesh(core_axis_name="c", subcore_axis_name="s")
    fn = functools.partial(
        pl.kernel,
        out_type=jax.ShapeDtypeStruct((ENT, SEQ, BATCH), jnp.float32),
        mesh=mesh,
        compiler_params=pltpu.CompilerParams(needs_layout_passes=False),
        scratch_types=[
            pltpu.VMEM((P_ROWS, LANES), jnp.float32),
            pltpu.VMEM((16, LANE_W), jnp.int32),
            pltpu.VMEM((16, LANE_W), jnp.int32),
            pltpu.VMEM((ENT, 16, LANE_W), jnp.float32),
            pltpu.VMEM((ENT, 16, LANE_W), jnp.float32),
            pltpu.SemaphoreType.DMA,
            pltpu.SemaphoreType.DMA,
            pltpu.SemaphoreType.DMA,
            pltpu.SemaphoreType.DMA,
        ],
    )(_sc_body)
    return fn(idx_t, params)


def kernel(inputs, emb_table, W, b):
    idx_t = inputs.astype(jnp.int32).T
    embT = jnp.pad(emb_table.T.astype(jnp.float32), ((0, 0), (0, LANES - VOCAB)))
    wrows = jnp.broadcast_to(
        W.T.astype(jnp.float32).reshape(ENT * EMB, 1), (ENT * EMB, LANES))
    brows = jnp.broadcast_to(b.astype(jnp.float32).reshape(ENT, 1), (ENT, LANES))
    params = jnp.concatenate([embT, wrows, brows], axis=0)
    out3 = _run(idx_t, params)
    return out3.transpose(2, 1, 0)
